# Initial kernel scaffold; baseline (speedup 1.0000x reference)
import jax
import jax.numpy as jnp
from jax.experimental import pallas as pl


def kernel(x, num_per_batch, degrees):
    raise NotImplementedError("write your pallas kernel here")


# speedup vs baseline: 6.3867x; 6.3867x over previous
import functools

import jax
import jax.numpy as jnp
from jax import lax
from jax.experimental import pallas as pl
from jax.experimental.pallas import tpu as pltpu
from jax.experimental.pallas import tpu_sc as plsc

TOTAL = 32768
D = 256
B = 16
DEGW = 2184
CH = 128
ACC_ROWS_PER_SEG = 16
NSEG_PER_SC = 8

_mesh = plsc.VectorSubcoreMesh(core_axis_name="c", subcore_axis_name="s")


def _body(x_hbm, nums_hbm, starts_hbm, degs_hbm, invs_hbm, out_hbm,
          deg_buf, xbuf, i2b, i4b, i8b, occ_tbl, nsv, stv, invv, zbuf, pbuf, acc):
    c = lax.axis_index("c")
    si = lax.axis_index("s")
    w = si * 2 + c

    @pl.when(w < B)
    def _():
        seg = w
        seg_local = si
        base_row = seg_local * ACC_ROWS_PER_SEG
        lane = lax.iota(jnp.int32, 16)

        pltpu.sync_copy(nums_hbm, nsv)
        pltpu.sync_copy(starts_hbm, stv)
        L = jnp.sum(jnp.where(lane == seg, nsv[...], 0))
        start = jnp.sum(jnp.where(lane == seg, stv[...], 0))
        k2 = (L + 1) >> 1
        k4 = (L + 3) >> 2
        k8 = (L + 7) >> 3

        b0 = pl.multiple_of(jnp.minimum(start & (-8), TOTAL - DEGW), 8)
        off = start - b0
        pltpu.sync_copy(degs_hbm.at[pl.ds(b0, DEGW)], deg_buf)

        zero16f = jnp.zeros((16,), jnp.float32)
        for r in range(ACC_ROWS_PER_SEG):
            for v in range(D // 16):
                zbuf[r, pl.ds(v * 16, 16)] = zero16f
        pltpu.sync_copy(zbuf, acc.at[pl.ds(base_row, ACC_ROWS_PER_SEG)])

        zero16i = jnp.zeros((16,), jnp.int32)
        for v in range(4):
            occ_tbl[pl.ds(v * 16, 16)] = zero16i
        nv = (L + 15) >> 4

        def h_body(v, carry):
            d = deg_buf[pl.ds(off + v * 16, 16)]
            valid = (v * 16 + lane) < L
            cnt, last = plsc.scan_count(d, valid)
            plsc.addupdate_scatter(occ_tbl, [d], cnt, mask=last)
            return carry

        lax.fori_loop(0, nv, h_body, 0)

        h = [occ_tbl[pl.ds(v * 16, 16)] for v in range(4)]
        t = [jnp.sum(hv) for hv in h]
        above = [t[1] + t[2] + t[3], t[2] + t[3], t[3], jnp.int32(0)]
        for v in range(4):
            occ_tbl[pl.ds(v * 16, 16)] = above[v] + (t[v] - plsc.cumsum(h[v]))

        nch = (L + CH - 1) >> 7
        trash = base_row + 15

        def c_body(ci, carry):
            gb = start + ci * CH
            gbase = jnp.minimum(gb, TOTAL - CH)
            sh = gb - gbase
            pltpu.sync_copy(x_hbm.at[pl.ds(gbase, CH)], xbuf)
            dstart = gbase - b0
            loc0 = gbase - start
            for sv in range(CH // 16):
                d = deg_buf[pl.ds(dstart + sv * 16, 16)]
                j = sv * 16 + lane
                l_ = loc0 + j
                valid = (j >= sh) & (l_ < L)
                cnt, last = plsc.scan_count(d, valid)
                rk0 = plsc.load_gather(occ_tbl, [d])
                rank = rk0 + cnt - 1
                plsc.addupdate_scatter(occ_tbl, [d], cnt, mask=last)
                b2 = (rank >= k2).astype(jnp.int32)
                b4 = ((rank >= k4).astype(jnp.int32)
                      + (rank >= 2 * k4).astype(jnp.int32)
                      + (rank >= 3 * k4).astype(jnp.int32))
                b8 = (rank >= k8).astype(jnp.int32)
                for m in range(2, 8):
                    b8 = b8 + (rank >= m * k8).astype(jnp.int32)
                i2b[pl.ds(sv * 16, 16)] = jnp.where(valid, base_row + b2, trash)
                i4b[pl.ds(sv * 16, 16)] = jnp.where(valid, base_row + 2 + b4, trash)
                i8b[pl.ds(sv * 16, 16)] = jnp.where(valid, base_row + 6 + b8, trash)
            pltpu.sync_copy(xbuf, acc.at[i2b], add=True)
            pltpu.sync_copy(xbuf, acc.at[i4b], add=True)
            pltpu.sync_copy(xbuf, acc.at[i8b], add=True)
            return carry

        lax.fori_loop(0, nch, c_body, 0)

        pltpu.sync_copy(acc.at[pl.ds(base_row, ACC_ROWS_PER_SEG)], zbuf)
        pltpu.sync_copy(invs_hbm, invv)
        zf = jnp.float32(0.0)
        invL = jnp.sum(jnp.where(lane == seg, invv[0, :], zf))
        inv2 = jnp.sum(jnp.where(lane == seg, invv[1, :], zf))
        inv4 = jnp.sum(jnp.where(lane == seg, invv[2, :], zf))
        inv8 = jnp.sum(jnp.where(lane == seg, invv[3, :], zf))
        for v in range(D // 16):
            sl = pl.ds(v * 16, 16)
            s2a = zbuf[0, sl]
            s2b = zbuf[1, sl]
            pbuf[0, sl] = (s2a + s2b) * invL
            pbuf[1, sl] = s2a * inv2
            pbuf[2, sl] = s2b * inv2
            for r in range(4):
                pbuf[3 + r, sl] = zbuf[2 + r, sl] * inv4
            for r in range(8):
                pbuf[7 + r, sl] = zbuf[6 + r, sl] * inv8
        pltpu.sync_copy(pbuf, out_hbm.at[pl.ds(seg * 15, 15)])


_pooling_kernel = functools.partial(
    pl.kernel,
    out_type=jax.ShapeDtypeStruct((B * 15, D), jnp.float32),
    mesh=_mesh,
    compiler_params=pltpu.CompilerParams(
        needs_layout_passes=False, use_tc_tiling_on_sc=False),
    scratch_types=[
        pltpu.VMEM((DEGW,), jnp.int32),
        pltpu.VMEM((CH, D), jnp.float32),
        pltpu.VMEM((CH,), jnp.int32),
        pltpu.VMEM((CH,), jnp.int32),
        pltpu.VMEM((CH,), jnp.int32),
        pltpu.VMEM((64,), jnp.int32),
        pltpu.VMEM((16,), jnp.int32),
        pltpu.VMEM((16,), jnp.int32),
        pltpu.VMEM((4, 16), jnp.float32),
        pltpu.VMEM((ACC_ROWS_PER_SEG, D), jnp.float32),
        pltpu.VMEM((15, D), jnp.float32),
        pltpu.VMEM_SHARED((NSEG_PER_SC * ACC_ROWS_PER_SEG, D), jnp.float32),
    ],
)(_body)


@jax.jit
def kernel(x, num_per_batch, degrees):
    nums = num_per_batch.astype(jnp.int32)
    starts = jnp.cumsum(nums) - nums
    lf = nums.astype(jnp.float32)
    invs = jnp.stack([
        1.0 / lf,
        1.0 / ((nums + 1) >> 1).astype(jnp.float32),
        1.0 / ((nums + 3) >> 2).astype(jnp.float32),
        1.0 / ((nums + 7) >> 3).astype(jnp.float32),
    ])
    pooled = _pooling_kernel(x, nums, starts, degrees.astype(jnp.int32), invs)
    return pooled.reshape(B, 15, D).transpose(0, 2, 1).reshape(B, D * 15)


# speedup vs baseline: 8.6680x; 1.3572x over previous
import functools

import jax
import jax.numpy as jnp
from jax import lax
from jax.experimental import pallas as pl
from jax.experimental.pallas import tpu as pltpu
from jax.experimental.pallas import tpu_sc as plsc

TOTAL = 32768
D = 256
B = 16
DEGW = 2184
CH = 128
ACC_ROWS_PER_SEG = 16
NSEG_PER_SC = 8

_mesh = plsc.VectorSubcoreMesh(core_axis_name="c", subcore_axis_name="s")


def _body(x_hbm, nums_hbm, starts_hbm, degs_hbm, invs_hbm, out_hbm,
          deg_buf, xbuf, i2b, i4b, i8b, h0t, h1t, occ_tbl, nsv, stv, invv,
          zbuf, pbuf, sem_g, sem_s, acc):
    c = lax.axis_index("c")
    si = lax.axis_index("s")
    seg = c * NSEG_PER_SC + (si >> 1)
    half = si & 1
    seg_local = si >> 1
    base_row = seg_local * ACC_ROWS_PER_SEG
    lane = lax.iota(jnp.int32, 16)

    pltpu.sync_copy(nums_hbm, nsv)
    pltpu.sync_copy(starts_hbm, stv)
    L = jnp.sum(jnp.where(lane == seg, nsv[...], 0))
    start = jnp.sum(jnp.where(lane == seg, stv[...], 0))
    k2 = (L + 1) >> 1
    k4 = (L + 3) >> 2
    k8 = (L + 7) >> 3
    Lh0 = (L + 1) >> 1

    b0 = pl.multiple_of(jnp.minimum(start & (-8), TOTAL - DEGW), 8)
    off = start - b0
    pltpu.sync_copy(degs_hbm.at[pl.ds(b0, DEGW)], deg_buf)

    zero16i = jnp.zeros((16,), jnp.int32)
    for v in range(4):
        h0t[pl.ds(v * 16, 16)] = zero16i
        h1t[pl.ds(v * 16, 16)] = zero16i
    nv = (L + 15) >> 4

    def h_body(v, carry):
        d = deg_buf[pl.ds(off + v * 16, 16)]
        l_ = v * 16 + lane
        in0 = l_ < Lh0
        in1 = jnp.logical_and(l_ >= Lh0, l_ < L)
        cnt0, last0 = plsc.scan_count(d, in0)
        plsc.addupdate_scatter(h0t, [d], cnt0, mask=last0)
        cnt1, last1 = plsc.scan_count(d, in1)
        plsc.addupdate_scatter(h1t, [d], cnt1, mask=last1)
        return carry

    lax.fori_loop(0, nv, h_body, 0)

    h0 = [h0t[pl.ds(v * 16, 16)] for v in range(4)]
    h1 = [h1t[pl.ds(v * 16, 16)] for v in range(4)]
    ht = [a + b for a, b in zip(h0, h1)]
    t = [jnp.sum(hv) for hv in ht]
    above = [t[1] + t[2] + t[3], t[2] + t[3], t[3], jnp.int32(0)]
    for v in range(4):
        occ_tbl[pl.ds(v * 16, 16)] = (above[v] + (t[v] - plsc.cumsum(ht[v]))
                                      + half * h0[v])

    @pl.when(half == 0)
    def _():
        zero16f = jnp.zeros((16,), jnp.float32)
        for r in range(ACC_ROWS_PER_SEG):
            for v in range(D // 16):
                zbuf[r, pl.ds(v * 16, 16)] = zero16f
        pltpu.sync_copy(zbuf, acc.at[pl.ds(base_row, ACC_ROWS_PER_SEG)])

    plsc.subcore_barrier()

    lc0 = half * Lh0
    lim = Lh0 + half * (L - Lh0)
    nch = (lim - lc0 + CH - 1) >> 7
    trash = base_row + 15

    def c_body(ci, carry):
        gb = start + lc0 + ci * CH
        gbase = jnp.minimum(gb, TOTAL - CH)
        sh = gb - gbase
        gdesc = pltpu.async_copy(x_hbm.at[pl.ds(gbase, CH)], xbuf, sem_g)
        dstart = gbase - b0
        loc0 = gbase - start
        for sv in range(CH // 16):
            d = deg_buf[pl.ds(dstart + sv * 16, 16)]
            j = sv * 16 + lane
            l_ = loc0 + j
            valid = (j >= sh) & (l_ < lim)
            cnt, last = plsc.scan_count(d, valid)
            rk0 = plsc.load_gather(occ_tbl, [d])
            rank = rk0 + cnt - 1
            plsc.addupdate_scatter(occ_tbl, [d], cnt, mask=last)
            b2 = (rank >= k2).astype(jnp.int32)
            b4 = ((rank >= k4).astype(jnp.int32)
                  + (rank >= 2 * k4).astype(jnp.int32)
                  + (rank >= 3 * k4).astype(jnp.int32))
            b8 = (rank >= k8).astype(jnp.int32)
            for m in range(2, 8):
                b8 = b8 + (rank >= m * k8).astype(jnp.int32)
            i2b[pl.ds(sv * 16, 16)] = jnp.where(valid, base_row + b2, trash)
            i4b[pl.ds(sv * 16, 16)] = jnp.where(valid, base_row + 2 + b4, trash)
            i8b[pl.ds(sv * 16, 16)] = jnp.where(valid, base_row + 6 + b8, trash)
        gdesc.wait()
        s2 = pltpu.async_copy(xbuf, acc.at[i2b], sem_s, add=True)
        s4 = pltpu.async_copy(xbuf, acc.at[i4b], sem_s, add=True)
        s8 = pltpu.async_copy(xbuf, acc.at[i8b], sem_s, add=True)
        s2.wait()
        s4.wait()
        s8.wait()
        return carry

    lax.fori_loop(0, nch, c_body, 0)

    plsc.subcore_barrier()

    @pl.when(half == 0)
    def _():
        pltpu.sync_copy(acc.at[pl.ds(base_row, ACC_ROWS_PER_SEG)], zbuf)
        pltpu.sync_copy(invs_hbm, invv)
        zf = jnp.float32(0.0)
        invL = jnp.sum(jnp.where(lane == seg, invv[0, :], zf))
        inv2 = jnp.sum(jnp.where(lane == seg, invv[1, :], zf))
        inv4 = jnp.sum(jnp.where(lane == seg, invv[2, :], zf))
        inv8 = jnp.sum(jnp.where(lane == seg, invv[3, :], zf))
        for v in range(D // 16):
            sl = pl.ds(v * 16, 16)
            s2a = zbuf[0, sl]
            s2b = zbuf[1, sl]
            pbuf[0, sl] = (s2a + s2b) * invL
            pbuf[1, sl] = s2a * inv2
            pbuf[2, sl] = s2b * inv2
            for r in range(4):
                pbuf[3 + r, sl] = zbuf[2 + r, sl] * inv4
            for r in range(8):
                pbuf[7 + r, sl] = zbuf[6 + r, sl] * inv8
        pltpu.sync_copy(pbuf, out_hbm.at[pl.ds(seg * 15, 15)])


_pooling_kernel = functools.partial(
    pl.kernel,
    out_type=jax.ShapeDtypeStruct((B * 15, D), jnp.float32),
    mesh=_mesh,
    compiler_params=pltpu.CompilerParams(
        needs_layout_passes=False, use_tc_tiling_on_sc=False),
    scratch_types=[
        pltpu.VMEM((DEGW,), jnp.int32),
        pltpu.VMEM((CH, D), jnp.float32),
        pltpu.VMEM((CH,), jnp.int32),
        pltpu.VMEM((CH,), jnp.int32),
        pltpu.VMEM((CH,), jnp.int32),
        pltpu.VMEM((64,), jnp.int32),
        pltpu.VMEM((64,), jnp.int32),
        pltpu.VMEM((64,), jnp.int32),
        pltpu.VMEM((16,), jnp.int32),
        pltpu.VMEM((16,), jnp.int32),
        pltpu.VMEM((4, 16), jnp.float32),
        pltpu.VMEM((ACC_ROWS_PER_SEG, D), jnp.float32),
        pltpu.VMEM((15, D), jnp.float32),
        pltpu.SemaphoreType.DMA,
        pltpu.SemaphoreType.DMA,
        pltpu.VMEM_SHARED((NSEG_PER_SC * ACC_ROWS_PER_SEG, D), jnp.float32),
    ],
)(_body)


@jax.jit
def kernel(x, num_per_batch, degrees):
    nums = num_per_batch.astype(jnp.int32)
    starts = jnp.cumsum(nums) - nums
    lf = nums.astype(jnp.float32)
    invs = jnp.stack([
        1.0 / lf,
        1.0 / ((nums + 1) >> 1).astype(jnp.float32),
        1.0 / ((nums + 3) >> 2).astype(jnp.float32),
        1.0 / ((nums + 7) >> 3).astype(jnp.float32),
    ])
    pooled = _pooling_kernel(x, nums, starts, degrees.astype(jnp.int32), invs)
    return pooled.reshape(B, 15, D).transpose(0, 2, 1).reshape(B, D * 15)
